# baseline (speedup 1.0000x reference)
import functools

import jax
import jax.numpy as jnp
from jax import lax
from jax.experimental import pallas as pl
from jax.experimental.pallas import tpu as pltpu
from jax.experimental.pallas import tpu_sc as plsc

NUM_EMB = 1000000
DIM = 64
B_TOKENS = 4096
SEQ = 200
B = B_TOKENS * SEQ

_info = plsc.get_sparse_core_info()
NC, NS, NL = _info.num_cores, _info.num_subcores, _info.num_lanes
NW = NC * NS
CHUNK = 128
NBT = B_TOKENS // CHUNK
NCHUNKS = SEQ * NBT
CH_PER_W = NCHUNKS // NW
NBUF = 4
NITER = CH_PER_W // NBUF


@functools.partial(
    pl.kernel,
    mesh=plsc.VectorSubcoreMesh(core_axis_name="c", subcore_axis_name="s"),
    out_type=jax.ShapeDtypeStruct((SEQ, DIM // 8, NBT, 8, CHUNK), jnp.float32),
    scratch_types=[
        pltpu.VMEM((CH_PER_W * CHUNK,), jnp.int32),
        *[pltpu.VMEM((CHUNK, DIM), jnp.float32) for _ in range(NBUF)],
        *[pltpu.VMEM((DIM // 8, 8, CHUNK + 1), jnp.float32) for _ in range(NBUF)],
        *[pltpu.SemaphoreType.DMA for _ in range(2 * NBUF)],
    ],
    compiler_params=pltpu.CompilerParams(use_tc_tiling_on_sc=False,
                                         needs_layout_passes=False),
)
def _gather_kernel(ids_hbm, table_hbm, out_hbm, idx_v, *bufs):
    rows = bufs[:NBUF]
    tiles = bufs[NBUF:2 * NBUF]
    gsem = bufs[2 * NBUF:3 * NBUF]
    wsem = bufs[3 * NBUF:4 * NBUF]

    wid = lax.axis_index("s") * NC + lax.axis_index("c")
    base_chunk = wid * CH_PER_W

    iota = jax.lax.iota(jnp.int32, 16)
    dt_vec = [(iota + 16 * g) // 8 for g in range(DIM // 16)]
    di_vec = iota % 8

    def gather(k, r, issue=True):
        mk = pltpu.async_copy if issue else pltpu.make_async_copy
        return mk(
            table_hbm.at[idx_v.at[pl.ds(k * CHUNK, CHUNK)]], rows[r], gsem[r])

    def write(k, r, issue=True):
        c = base_chunk + k
        s = c // NBT
        bt = c % NBT
        mk = pltpu.async_copy if issue else pltpu.make_async_copy
        return mk(tiles[r].at[:, :, pl.ds(0, CHUNK)], out_hbm.at[s, :, bt],
                  wsem[r])

    def transpose(r):
        U = 8
        NG = DIM // 16

        def b_body(j, carry):
            b0 = j * U
            bb = [jnp.zeros((16,), jnp.int32) + (b0 + u) for u in range(U)]
            vs = [[rows[r][b0 + u, pl.ds(16 * g, 16)] for g in range(NG)]
                  for u in range(U)]
            for u in range(U):
                for g in range(NG):
                    plsc.store_scatter(tiles[r], [dt_vec[g], di_vec, bb[u]],
                                       vs[u][g])
            return carry

        lax.fori_loop(0, CHUNK // U, b_body, 0)

    pltpu.sync_copy(ids_hbm.at[pl.ds(base_chunk * CHUNK, CH_PER_W * CHUNK)],
                    idx_v)
    for r in range(NBUF):
        gather(r, r)

    def ring_body(i, carry):
        for r in range(NBUF):
            k = i * NBUF + r
            gather(k, r, issue=False).wait()

            @pl.when(i > 0)
            def _():
                write(k - NBUF, r, issue=False).wait()

            transpose(r)
            write(k, r)

            @pl.when(i < NITER - 1)
            def _():
                gather(k + NBUF, r)

        return carry

    lax.fori_loop(0, NITER, ring_body, 0)
    for r in range(NBUF):
        write(CH_PER_W - NBUF + r, r, issue=False).wait()


def kernel(token_ids, emb_mat):
    flat_ids = token_ids.T.astype(jnp.int32).reshape(B)
    out5 = _gather_kernel(flat_ids, emb_mat)
    return out5.transpose(2, 4, 0, 1, 3).reshape(B_TOKENS, SEQ, DIM)


# speedup vs baseline: 1.0090x; 1.0090x over previous
import functools

import jax
import jax.numpy as jnp
from jax import lax
from jax.experimental import pallas as pl
from jax.experimental.pallas import tpu as pltpu
from jax.experimental.pallas import tpu_sc as plsc

NUM_EMB = 1000000
DIM = 64
B_TOKENS = 4096
SEQ = 200
B = B_TOKENS * SEQ

_info = plsc.get_sparse_core_info()
NC, NS, NL = _info.num_cores, _info.num_subcores, _info.num_lanes
NW = NC * NS
CHUNK = 128
NBT = B_TOKENS // CHUNK
NCHUNKS = SEQ * NBT
CH_PER_W = NCHUNKS // NW
NBUF = 5
NITER = CH_PER_W // NBUF


@functools.partial(
    pl.kernel,
    mesh=plsc.VectorSubcoreMesh(core_axis_name="c", subcore_axis_name="s"),
    out_type=jax.ShapeDtypeStruct((SEQ, DIM // 8, NBT, 8, CHUNK), jnp.float32),
    scratch_types=[
        pltpu.VMEM((CH_PER_W * CHUNK,), jnp.int32),
        *[pltpu.VMEM((CHUNK, DIM), jnp.float32) for _ in range(NBUF)],
        *[pltpu.VMEM((DIM // 8, 8, CHUNK + 1), jnp.float32) for _ in range(NBUF)],
        *[pltpu.SemaphoreType.DMA for _ in range(2 * NBUF)],
    ],
    compiler_params=pltpu.CompilerParams(use_tc_tiling_on_sc=False,
                                         needs_layout_passes=False),
)
def _gather_kernel(ids_hbm, table_hbm, out_hbm, idx_v, *bufs):
    rows = bufs[:NBUF]
    tiles = bufs[NBUF:2 * NBUF]
    gsem = bufs[2 * NBUF:3 * NBUF]
    wsem = bufs[3 * NBUF:4 * NBUF]

    wid = lax.axis_index("s") * NC + lax.axis_index("c")
    base_chunk = wid * CH_PER_W

    iota = jax.lax.iota(jnp.int32, 16)
    dt_vec = [(iota + 16 * g) // 8 for g in range(DIM // 16)]
    di_vec = iota % 8

    def gather(k, r, issue=True):
        mk = pltpu.async_copy if issue else pltpu.make_async_copy
        return mk(
            table_hbm.at[idx_v.at[pl.ds(k * CHUNK, CHUNK)]], rows[r], gsem[r])

    def write(k, r, issue=True):
        c = base_chunk + k
        s = c // NBT
        bt = c % NBT
        mk = pltpu.async_copy if issue else pltpu.make_async_copy
        return mk(tiles[r].at[:, :, pl.ds(0, CHUNK)], out_hbm.at[s, :, bt],
                  wsem[r])

    def transpose(r):
        U = 4
        NG = DIM // 16

        def b_body(j, carry):
            b0 = j * U
            bb = [jnp.zeros((16,), jnp.int32) + (b0 + u) for u in range(U)]
            vs = [[rows[r][b0 + u, pl.ds(16 * g, 16)] for g in range(NG)]
                  for u in range(U)]
            for u in range(U):
                for g in range(NG):
                    plsc.store_scatter(tiles[r], [dt_vec[g], di_vec, bb[u]],
                                       vs[u][g])
            return carry

        lax.fori_loop(0, CHUNK // U, b_body, 0)

    pltpu.sync_copy(ids_hbm.at[pl.ds(base_chunk * CHUNK, CH_PER_W * CHUNK)],
                    idx_v)
    for r in range(NBUF):
        gather(r, r)

    def ring_body(i, carry):
        for r in range(NBUF):
            k = i * NBUF + r
            gather(k, r, issue=False).wait()

            @pl.when(i > 0)
            def _():
                write(k - NBUF, r, issue=False).wait()

            transpose(r)
            write(k, r)

            @pl.when(i < NITER - 1)
            def _():
                gather(k + NBUF, r)

        return carry

    lax.fori_loop(0, NITER, ring_body, 0)
    for r in range(NBUF):
        write(CH_PER_W - NBUF + r, r, issue=False).wait()


def kernel(token_ids, emb_mat):
    flat_ids = token_ids.T.astype(jnp.int32).reshape(B)
    out5 = _gather_kernel(flat_ids, emb_mat)
    return out5.transpose(2, 4, 0, 1, 3).reshape(B_TOKENS, SEQ, DIM)
